# baseline (speedup 1.0000x reference)
import jax
import jax.numpy as jnp
from jax.experimental import pallas as pl
from jax.experimental.pallas import tpu as pltpu

D_IN = 768
D_HIDDEN = 8192
K = 64
N_TOK = 2048

TB = 128
TB_DEC = 256


def _encode_topk_kernel(x_ref, w_hbm, be_ref, o_ref, w_vmem, sem):
    i = pl.program_id(0)

    @pl.when(i == 0)
    def _():
        cp = pltpu.make_async_copy(w_hbm, w_vmem, sem)
        cp.start()
        cp.wait()

    enc = jax.lax.dot_general(
        x_ref[...], w_vmem[...], (((1,), (1,)), ((), ())),
        preferred_element_type=jnp.float32)
    s = jnp.maximum(enc + be_ref[...], 0.0)
    si = jax.lax.bitcast_convert_type(s, jnp.int32)

    NCHAIN = 4
    ROWS = TB // NCHAIN
    sis = [si[c * ROWS:(c + 1) * ROWS, :] for c in range(NCHAIN)]

    def init(_):
        return (jnp.zeros((ROWS, 1), jnp.int32),
                jnp.full((ROWS, 1), jnp.int32(0x7F800000)),
                jnp.zeros((ROWS, 1), jnp.int32),
                jnp.zeros((ROWS, 1), jnp.int32))

    def cond(carry):
        it, _, alldone = carry
        return jnp.logical_and(it < 31, jnp.logical_not(alldone))

    def body(carry):
        it, chains, _ = carry
        new_chains = []
        mins = []
        for c in range(NCHAIN):
            lo, hi, thr, done = chains[c]
            mid = lo + (hi - lo) // 2
            neg = jax.lax.shift_right_arithmetic(sis[c] - mid, 31)
            cnt = jnp.sum(neg, axis=1, keepdims=True) + D_HIDDEN
            pred = cnt >= K
            lo = jnp.where(pred, mid, lo)
            hi = jnp.where(pred, hi, mid)
            newly = jnp.logical_and(cnt == K, done == 0)
            thr = jnp.where(newly, mid, thr)
            done = jnp.where(newly, 1, done)
            new_chains.append((lo, hi, thr, done))
            mins.append(jnp.min(done) == 1)
        alldone = mins[0]
        for m in mins[1:]:
            alldone = jnp.logical_and(alldone, m)
        return it + 1, tuple(new_chains), alldone

    _, chains, _ = jax.lax.while_loop(
        cond, body, (0, tuple(init(c) for c in range(NCHAIN)), False))

    thr_full = jnp.concatenate(
        [jnp.where(done == 1, thr, lo) for (lo, _, thr, done) in chains],
        axis=0)
    o_ref[...] = jnp.where(si >= thr_full, s, 0.0)


def _decode_kernel(s_ref, w_hbm, bd_ref, o_ref, w_vmem, sem):
    @pl.when(pl.program_id(0) == 0)
    def _():
        cp = pltpu.make_async_copy(w_hbm, w_vmem, sem)
        cp.start()
        cp.wait()

    o_ref[...] = (
        jnp.dot(s_ref[...], w_vmem[...], preferred_element_type=jnp.float32)
        + bd_ref[...]
    )


@jax.jit
def kernel(x, W_enc, b_enc, b_dec):
    n = x.shape[0]

    sparse = pl.pallas_call(
        _encode_topk_kernel,
        grid=(n // TB,),
        in_specs=[
            pl.BlockSpec((TB, D_IN), lambda i: (i, 0)),
            pl.BlockSpec(memory_space=pl.ANY),
            pl.BlockSpec((1, D_HIDDEN), lambda i: (0, 0)),
        ],
        out_specs=pl.BlockSpec((TB, D_HIDDEN), lambda i: (i, 0)),
        out_shape=jax.ShapeDtypeStruct((n, D_HIDDEN), jnp.float32),
        scratch_shapes=[
            pltpu.VMEM((D_HIDDEN, D_IN), jnp.float32),
            pltpu.SemaphoreType.DMA,
        ],
    )(x, W_enc, b_enc.reshape(1, D_HIDDEN))

    out = pl.pallas_call(
        _decode_kernel,
        grid=(n // TB_DEC,),
        in_specs=[
            pl.BlockSpec((TB_DEC, D_HIDDEN), lambda i: (i, 0)),
            pl.BlockSpec(memory_space=pl.ANY),
            pl.BlockSpec((1, D_IN), lambda i: (0, 0)),
        ],
        out_specs=pl.BlockSpec((TB_DEC, D_IN), lambda i: (i, 0)),
        out_shape=jax.ShapeDtypeStruct((n, D_IN), jnp.float32),
        scratch_shapes=[
            pltpu.VMEM((D_HIDDEN, D_IN), jnp.float32),
            pltpu.SemaphoreType.DMA,
        ],
    )(sparse, W_enc, b_dec.reshape(1, D_IN))

    return out


# speedup vs baseline: 1.0658x; 1.0658x over previous
import jax
import jax.numpy as jnp
from jax.experimental import pallas as pl
from jax.experimental.pallas import tpu as pltpu

D_IN = 768
D_HIDDEN = 8192
K = 64
N_TOK = 2048

TB = 128
TB_DEC = 256


def _encode_topk_kernel(x_ref, w_hbm, be_ref, bd_ref, o_ref, w_vmem, sem):
    i = pl.program_id(0)

    @pl.when(i == 0)
    def _():
        cp = pltpu.make_async_copy(w_hbm, w_vmem, sem)
        cp.start()
        cp.wait()

    enc = jax.lax.dot_general(
        x_ref[...], w_vmem[...], (((1,), (1,)), ((), ())),
        preferred_element_type=jnp.float32)
    s = jnp.maximum(enc + be_ref[...], 0.0)
    si = jax.lax.bitcast_convert_type(s, jnp.int32)

    NCHAIN = 2
    ROWS = TB // NCHAIN
    sis = [si[c * ROWS:(c + 1) * ROWS, :] for c in range(NCHAIN)]

    def init(_):
        return (jnp.zeros((ROWS, 1), jnp.int32),
                jnp.full((ROWS, 1), jnp.int32(0x7F800000)),
                jnp.zeros((ROWS, 1), jnp.int32),
                jnp.zeros((ROWS, 1), jnp.int32))

    def cond(carry):
        it, _, alldone = carry
        return jnp.logical_and(it < 31, jnp.logical_not(alldone))

    def body(carry):
        it, chains, _ = carry
        new_chains = []
        mins = []
        for c in range(NCHAIN):
            lo, hi, thr, done = chains[c]
            mid = lo + (hi - lo) // 2
            neg = jax.lax.shift_right_arithmetic(sis[c] - mid, 31)
            cnt = jnp.sum(neg, axis=1, keepdims=True) + D_HIDDEN
            pred = cnt >= K
            lo = jnp.where(pred, mid, lo)
            hi = jnp.where(pred, hi, mid)
            newly = jnp.logical_and(cnt == K, done == 0)
            thr = jnp.where(newly, mid, thr)
            done = jnp.where(newly, 1, done)
            new_chains.append((lo, hi, thr, done))
            mins.append(jnp.min(done) == 1)
        alldone = mins[0]
        for m in mins[1:]:
            alldone = jnp.logical_and(alldone, m)
        return it + 1, tuple(new_chains), alldone

    _, chains, _ = jax.lax.while_loop(
        cond, body, (0, tuple(init(c) for c in range(NCHAIN)), False))

    thr_full = jnp.concatenate(
        [jnp.where(done == 1, thr, lo) for (lo, _, thr, done) in chains],
        axis=0)
    sparse = jnp.where(si >= thr_full, s, 0.0)

    o_ref[...] = (
        jnp.dot(sparse, w_vmem[...], preferred_element_type=jnp.float32)
        + bd_ref[...]
    )


@jax.jit
def kernel(x, W_enc, b_enc, b_dec):
    n = x.shape[0]

    out = pl.pallas_call(
        _encode_topk_kernel,
        grid=(n // TB,),
        in_specs=[
            pl.BlockSpec((TB, D_IN), lambda i: (i, 0)),
            pl.BlockSpec(memory_space=pl.ANY),
            pl.BlockSpec((1, D_HIDDEN), lambda i: (0, 0)),
            pl.BlockSpec((1, D_IN), lambda i: (0, 0)),
        ],
        out_specs=pl.BlockSpec((TB, D_IN), lambda i: (i, 0)),
        out_shape=jax.ShapeDtypeStruct((n, D_IN), jnp.float32),
        scratch_shapes=[
            pltpu.VMEM((D_HIDDEN, D_IN), jnp.float32),
            pltpu.SemaphoreType.DMA,
        ],
    )(x, W_enc, b_enc.reshape(1, D_HIDDEN), b_dec.reshape(1, D_IN))

    return out
